# baseline (speedup 1.0000x reference)
import functools

import jax
import jax.numpy as jnp
from jax.experimental import pallas as pl
from jax.experimental.pallas import tpu as pltpu

_GRID_SIZE = 16.0
_MARGIN = 1.0
_NUM_NEG = 8
_SOS_NEG = 8
_N = 1024
_C = 256
_M = 1024
_BIG = 1.0e6
_RADIUS = _GRID_SIZE * (2.0 ** 0.5) + 0.1


def _row_min(x):
    return jnp.min(x, axis=1, keepdims=True)


def _treemin(xs):
    while len(xs) > 1:
        xs = [jnp.minimum(xs[i], xs[i + 1]) for i in range(0, len(xs) - 1, 2)
              ] + (xs[-1:] if len(xs) % 2 else [])
    return xs[0]


def _top4_axis(vals):
    out_v, out_i = [], []
    for _ in range(4):
        m = _treemin(vals)
        f = _treemin([jnp.where(v == m, jnp.float32(j), jnp.float32(1e9))
                      for j, v in enumerate(vals)])
        vals = [jnp.where((v == m) & (f == jnp.float32(j)),
                          jnp.float32(jnp.inf), v)
                for j, v in enumerate(vals)]
        out_v.append(m)
        out_i.append(f)
    return out_v, out_i


def _top4_cells_packed(px, py):
    dx2 = [(px - (j + 0.5) * _GRID_SIZE) ** 2 for j in range(32)]
    dy2 = [(py - (i + 0.5) * _GRID_SIZE) ** 2 for i in range(32)]
    vx, jx = _top4_axis(dx2)
    vy, iy = _top4_axis(dy2)
    ds = [vy[a] + vx[b] for a in range(4) for b in range(4)]
    fl = [iy[a] * 32.0 + jx[b] for a in range(4) for b in range(4)]
    out = []
    for _ in range(4):
        m = _treemin(ds)
        f = _treemin([jnp.where(d == m, g, jnp.float32(1e9))
                      for d, g in zip(ds, fl)])
        ds = [jnp.where((d == m) & (g == f), jnp.float32(jnp.inf), d)
              for d, g in zip(ds, fl)]
        out.append(f)
    return out


def _extract_min(x, cols_f):
    minv = _row_min(x)
    cand = jnp.where(x == minv, cols_f, jnp.float32(2.0 * _M))
    amin = _row_min(cand)
    x = jnp.where(cols_f == amin, jnp.float32(jnp.inf), x)
    return minv, amin, x


def _loss_kernel(kxc_ref, kyc_ref, kxr_ref, kyr_ref,
                 wxc_ref, wyc_ref, wxr_ref, wyr_ref,
                 kxp_ref, kyp_ref,
                 desc_ref, d2r_ref, homo_ref, out_ref):
    b = pl.program_id(0)

    @pl.when(b == 0)
    def _init():
        out_ref[0, 0] = jnp.float32(0.0)

    kxc = kxc_ref[0]
    kyc = kyc_ref[0]
    kxr = kxr_ref[0]
    kyr = kyr_ref[0]
    wxc = wxc_ref[0]
    wyc = wyc_ref[0]
    wxr = wxr_ref[0]
    wyr = wyr_ref[0]
    desc = desc_ref[0]
    d2r = d2r_ref[0]

    cols_f = jax.lax.broadcasted_iota(jnp.int32, (1, _M), 1).astype(
        jnp.float32)
    cols3 = jax.lax.broadcasted_iota(jnp.int32, (1, 1, _M), 2).astype(
        jnp.float32)
    cell_i = jnp.floor(cols_f * (1.0 / 32.0))
    cell_j = cols_f - 32.0 * cell_i
    cellx = (cell_j + 0.5) * _GRID_SIZE
    celly = (cell_i + 0.5) * _GRID_SIZE

    x = jnp.clip(wxc * (1.0 / _GRID_SIZE) - 0.5, 0.0, 31.0)
    y = jnp.clip(wyc * (1.0 / _GRID_SIZE) - 0.5, 0.0, 31.0)
    x0 = jnp.floor(x)
    y0 = jnp.floor(y)
    x1 = jnp.minimum(x0 + 1.0, 31.0)
    y1 = jnp.minimum(y0 + 1.0, 31.0)
    wx = x - x0
    wy = y - y0
    w00 = (1.0 - wy) * (1.0 - wx)
    w01 = (1.0 - wy) * wx
    w10 = wy * (1.0 - wx)
    w11 = wy * wx
    onehot = (w00 * (cols_f == y0 * 32.0 + x0) +
              w01 * (cols_f == y0 * 32.0 + x1) +
              w10 * (cols_f == y1 * 32.0 + x0) +
              w11 * (cols_f == y1 * 32.0 + x1))
    wdesc = jax.lax.dot_general(onehot, d2r, (((1,), (0,)), ((), ())),
                                preferred_element_type=jnp.float32)
    nrm = jnp.sqrt(jnp.sum(wdesc * wdesc, axis=1, keepdims=True))
    wdesc = wdesc / (nrm + 1e-8)

    pos = jnp.sqrt(jnp.clip(2.0 - 2.0 * jnp.sum(desc * wdesc, axis=1,
                                                keepdims=True), 1e-8))

    desc_sim2 = 2.0 - 2.0 * jax.lax.dot_general(
        desc, d2r, (((1,), (1,)), ((), ())),
        preferred_element_type=jnp.float32)

    h00 = homo_ref[0, 0, 0]
    h01 = homo_ref[0, 0, 1]
    h02 = homo_ref[0, 0, 2]
    h10 = homo_ref[0, 0, 3]
    h11 = homo_ref[0, 0, 4]
    h12 = homo_ref[0, 0, 5]
    h20 = homo_ref[0, 0, 6]
    h21 = homo_ref[0, 0, 7]
    h22 = homo_ref[0, 0, 8]

    kxp = kxp_ref[0]
    kyp = kyp_ref[0]
    for f in _top4_cells_packed(kxp, kyp):
        ci = jnp.floor(f * (1.0 / 32.0))
        cj = f - 32.0 * ci
        cx = (cj + 0.5) * _GRID_SIZE
        cy = (ci + 0.5) * _GRID_SIZE
        den = h20 * cx + h21 * cy + h22 + 1e-8
        wcx = (h00 * cx + h01 * cy + h02) / den
        wcy = (h10 * cx + h11 * cy + h12) / den
        for f2 in _top4_cells_packed(wcx, wcy):
            sim3 = desc_sim2.reshape(8, 128, _M)
            sim3 = sim3 + _BIG * (cols3 == f2[:, :, None])
            desc_sim2 = sim3.reshape(_N, _M)

    fos_vec = jnp.zeros((_N, 1), jnp.float32)
    xs = desc_sim2
    for _k in range(_NUM_NEG):
        minv, _, xs = _extract_min(xs, cols_f)
        neg = jnp.sqrt(jnp.clip(minv, 1e-8))
        fos_vec = fos_vec + jnp.clip(pos - neg + _MARGIN, 0.0) ** 2
    fos_sum = jnp.sum(fos_vec)

    kp1_sim2 = 2.0 - 2.0 * jax.lax.dot_general(
        desc, desc, (((1,), (1,)), ((), ())),
        preferred_element_type=jnp.float32)
    kdist = jnp.sqrt((kxc - kxr) ** 2 + (kyc - kyr) ** 2 + 1e-8)
    kp1_sim2 = kp1_sim2 + _BIG * (kdist <= _RADIUS)

    w_sim2 = 2.0 - 2.0 * jax.lax.dot_general(
        wdesc, wdesc, (((1,), (1,)), ((), ())),
        preferred_element_type=jnp.float32)
    wdist = jnp.sqrt((wxc - wxr) ** 2 + (wyc - wyr) ** 2 + 1e-8)
    w_sim2 = w_sim2 + _BIG * (wdist <= _RADIUS)

    colsn_f = jax.lax.broadcasted_iota(jnp.int32, (1, _N), 1).astype(
        jnp.float32)
    sos_vec = jnp.zeros((_N, 1), jnp.float32)
    for _k in range(_SOS_NEG):
        mva, _, kp1_sim2 = _extract_min(kp1_sim2, colsn_f)
        mvb, _, w_sim2 = _extract_min(w_sim2, colsn_f)
        a = jnp.sqrt(jnp.clip(mva, 1e-8))
        bb = jnp.sqrt(jnp.clip(mvb, 1e-8))
        sos_vec = sos_vec + (a - bb) ** 2
    sos_sum = jnp.sum(jnp.sqrt(sos_vec + 1e-8))

    contrib = fos_sum / (2.0 * _N * _NUM_NEG) + sos_sum / (2.0 * _N)
    out_ref[0, 0] += contrib


@jax.jit
def kernel(kp1, w_kp1, kp1_desc, desc2, homo12):
    b = kp1.shape[0]
    kxc = kp1[..., 0].reshape(b, _N, 1)
    kyc = kp1[..., 1].reshape(b, _N, 1)
    kxr = kp1[..., 0].reshape(b, 1, _N)
    kyr = kp1[..., 1].reshape(b, 1, _N)
    wxc = w_kp1[..., 0].reshape(b, _N, 1)
    wyc = w_kp1[..., 1].reshape(b, _N, 1)
    wxr = w_kp1[..., 0].reshape(b, 1, _N)
    wyr = w_kp1[..., 1].reshape(b, 1, _N)
    kxp = kp1[..., 0].reshape(b, 8, 128)
    kyp = kp1[..., 1].reshape(b, 8, 128)
    d2r = jnp.transpose(desc2, (0, 2, 3, 1)).reshape(b, _M, _C)
    homo = homo12.reshape(b, 1, 9)

    col3 = pl.BlockSpec((1, _N, 1), lambda i: (i, 0, 0))
    row3 = pl.BlockSpec((1, 1, _N), lambda i: (i, 0, 0))

    out = pl.pallas_call(
        _loss_kernel,
        grid=(b,),
        in_specs=[
            col3, col3, row3, row3,
            col3, col3, row3, row3,
            pl.BlockSpec((1, 8, 128), lambda i: (i, 0, 0)),
            pl.BlockSpec((1, 8, 128), lambda i: (i, 0, 0)),
            pl.BlockSpec((1, _N, _C), lambda i: (i, 0, 0)),
            pl.BlockSpec((1, _M, _C), lambda i: (i, 0, 0)),
            pl.BlockSpec((1, 1, 9), lambda i: (i, 0, 0),
                         memory_space=pltpu.SMEM),
        ],
        out_specs=pl.BlockSpec((1, 1), lambda i: (0, 0),
                               memory_space=pltpu.SMEM),
        out_shape=jax.ShapeDtypeStruct((1, 1), jnp.float32),
    )(kxc, kyc, kxr, kyr, wxc, wyc, wxr, wyr, kxp, kyp, kp1_desc, d2r, homo)
    return out[0, 0]


# speedup vs baseline: 1.7672x; 1.7672x over previous
import functools

import jax
import jax.numpy as jnp
from jax.experimental import pallas as pl
from jax.experimental.pallas import tpu as pltpu

_GRID_SIZE = 16.0
_MARGIN = 1.0
_NUM_NEG = 8
_SOS_NEG = 8
_N = 1024
_C = 256
_M = 1024
_BIG = 1.0e6
_RADIUS = _GRID_SIZE * (2.0 ** 0.5) + 0.1


def _row_min(x):
    return jnp.min(x, axis=1, keepdims=True)


def _treemin(xs):
    while len(xs) > 1:
        xs = [jnp.minimum(xs[i], xs[i + 1]) for i in range(0, len(xs) - 1, 2)
              ] + (xs[-1:] if len(xs) % 2 else [])
    return xs[0]


def _top4_axis(vals):
    out_v, out_i = [], []
    for _ in range(4):
        m = _treemin(vals)
        f = _treemin([jnp.where(v == m, jnp.float32(j), jnp.float32(1e9))
                      for j, v in enumerate(vals)])
        vals = [jnp.where((v == m) & (f == jnp.float32(j)),
                          jnp.float32(jnp.inf), v)
                for j, v in enumerate(vals)]
        out_v.append(m)
        out_i.append(f)
    return out_v, out_i


def _top4_cells_packed(px, py):
    dx2 = [(px - (j + 0.5) * _GRID_SIZE) ** 2 for j in range(32)]
    dy2 = [(py - (i + 0.5) * _GRID_SIZE) ** 2 for i in range(32)]
    vx, jx = _top4_axis(dx2)
    vy, iy = _top4_axis(dy2)
    ds = [vy[a] + vx[b] for a in range(4) for b in range(4)]
    fl = [iy[a] * 32.0 + jx[b] for a in range(4) for b in range(4)]
    out = []
    for _ in range(4):
        m = _treemin(ds)
        f = _treemin([jnp.where(d == m, g, jnp.float32(1e9))
                      for d, g in zip(ds, fl)])
        ds = [jnp.where((d == m) & (g == f), jnp.float32(jnp.inf), d)
              for d, g in zip(ds, fl)]
        out.append(f)
    return out


def _extract_min(x, cols_f):
    minv = _row_min(x)
    cand = jnp.where(x == minv, cols_f, jnp.float32(2.0 * _M))
    amin = _row_min(cand)
    x = jnp.where(cols_f == amin, jnp.float32(jnp.inf), x)
    return minv, amin, x


def _loss_kernel(kxc_ref, kyc_ref, kxr_ref, kyr_ref,
                 wxc_ref, wyc_ref, wxr_ref, wyr_ref,
                 kxp_ref, kyp_ref,
                 desc_ref, d2r_ref, homo_ref, out_ref):
    b = pl.program_id(0)

    @pl.when(b == 0)
    def _init():
        out_ref[0, 0] = jnp.float32(0.0)

    kxc = kxc_ref[0]
    kyc = kyc_ref[0]
    kxr = kxr_ref[0]
    kyr = kyr_ref[0]
    wxc = wxc_ref[0]
    wyc = wyc_ref[0]
    wxr = wxr_ref[0]
    wyr = wyr_ref[0]
    desc = desc_ref[0]
    d2r = d2r_ref[0]

    cols_f = jax.lax.broadcasted_iota(jnp.int32, (1, _M), 1).astype(
        jnp.float32)
    cols3 = jax.lax.broadcasted_iota(jnp.int32, (1, 1, _M), 2).astype(
        jnp.float32)
    cell_i = jnp.floor(cols_f * (1.0 / 32.0))
    cell_j = cols_f - 32.0 * cell_i
    cellx = (cell_j + 0.5) * _GRID_SIZE
    celly = (cell_i + 0.5) * _GRID_SIZE

    x = jnp.clip(wxc * (1.0 / _GRID_SIZE) - 0.5, 0.0, 31.0)
    y = jnp.clip(wyc * (1.0 / _GRID_SIZE) - 0.5, 0.0, 31.0)
    x0 = jnp.floor(x)
    y0 = jnp.floor(y)
    x1 = jnp.minimum(x0 + 1.0, 31.0)
    y1 = jnp.minimum(y0 + 1.0, 31.0)
    wx = x - x0
    wy = y - y0
    w00 = (1.0 - wy) * (1.0 - wx)
    w01 = (1.0 - wy) * wx
    w10 = wy * (1.0 - wx)
    w11 = wy * wx
    onehot = (w00 * (cols_f == y0 * 32.0 + x0) +
              w01 * (cols_f == y0 * 32.0 + x1) +
              w10 * (cols_f == y1 * 32.0 + x0) +
              w11 * (cols_f == y1 * 32.0 + x1))
    wdesc = jax.lax.dot_general(onehot, d2r, (((1,), (0,)), ((), ())),
                                preferred_element_type=jnp.float32)
    nrm = jnp.sqrt(jnp.sum(wdesc * wdesc, axis=1, keepdims=True))
    wdesc = wdesc / (nrm + 1e-8)

    pos = jnp.sqrt(jnp.clip(2.0 - 2.0 * jnp.sum(desc * wdesc, axis=1,
                                                keepdims=True), 1e-8))

    desc_sim2 = 2.0 - 2.0 * jax.lax.dot_general(
        desc, d2r, (((1,), (1,)), ((), ())),
        preferred_element_type=jnp.float32)

    h00 = homo_ref[0, 0, 0]
    h01 = homo_ref[0, 0, 1]
    h02 = homo_ref[0, 0, 2]
    h10 = homo_ref[0, 0, 3]
    h11 = homo_ref[0, 0, 4]
    h12 = homo_ref[0, 0, 5]
    h20 = homo_ref[0, 0, 6]
    h21 = homo_ref[0, 0, 7]
    h22 = homo_ref[0, 0, 8]

    kxp = kxp_ref[0]
    kyp = kyp_ref[0]
    mask_ids = []
    for f in _top4_cells_packed(kxp, kyp):
        ci = jnp.floor(f * (1.0 / 32.0))
        cj = f - 32.0 * ci
        cx = (cj + 0.5) * _GRID_SIZE
        cy = (ci + 0.5) * _GRID_SIZE
        den = h20 * cx + h21 * cy + h22 + 1e-8
        wcx = (h00 * cx + h01 * cy + h02) / den
        wcy = (h10 * cx + h11 * cy + h12) / den
        mask_ids.extend(_top4_cells_packed(wcx, wcy))
    macc = [(cols3 == f2[:, :, None]).astype(jnp.float32) for f2 in mask_ids]
    while len(macc) > 1:
        macc = [macc[i] + macc[i + 1] for i in range(0, len(macc), 2)]
    desc_sim2 = (desc_sim2.reshape(8, 128, _M)
                 + _BIG * macc[0]).reshape(_N, _M)

    fos_vec = jnp.zeros((_N, 1), jnp.float32)
    xs = desc_sim2
    for _k in range(_NUM_NEG):
        minv = _row_min(xs)
        xs = jnp.where(xs == minv, jnp.float32(jnp.inf), xs)
        neg = jnp.sqrt(jnp.clip(minv, 1e-8))
        fos_vec = fos_vec + jnp.clip(pos - neg + _MARGIN, 0.0) ** 2
    fos_sum = jnp.sum(fos_vec)

    kp1_sim2 = 2.0 - 2.0 * jax.lax.dot_general(
        desc, desc, (((1,), (1,)), ((), ())),
        preferred_element_type=jnp.float32)
    kdist = jnp.sqrt((kxc - kxr) ** 2 + (kyc - kyr) ** 2 + 1e-8)
    kp1_sim2 = kp1_sim2 + _BIG * (kdist <= _RADIUS)

    w_sim2 = 2.0 - 2.0 * jax.lax.dot_general(
        wdesc, wdesc, (((1,), (1,)), ((), ())),
        preferred_element_type=jnp.float32)
    wdist = jnp.sqrt((wxc - wxr) ** 2 + (wyc - wyr) ** 2 + 1e-8)
    w_sim2 = w_sim2 + _BIG * (wdist <= _RADIUS)

    colsn_f = jax.lax.broadcasted_iota(jnp.int32, (1, _N), 1).astype(
        jnp.float32)
    sos_vec = jnp.zeros((_N, 1), jnp.float32)
    for _k in range(_SOS_NEG):
        mva = _row_min(kp1_sim2)
        kp1_sim2 = jnp.where(kp1_sim2 == mva, jnp.float32(jnp.inf), kp1_sim2)
        mvb = _row_min(w_sim2)
        w_sim2 = jnp.where(w_sim2 == mvb, jnp.float32(jnp.inf), w_sim2)
        a = jnp.sqrt(jnp.clip(mva, 1e-8))
        bb = jnp.sqrt(jnp.clip(mvb, 1e-8))
        sos_vec = sos_vec + (a - bb) ** 2
    sos_sum = jnp.sum(jnp.sqrt(sos_vec + 1e-8))

    contrib = fos_sum / (2.0 * _N * _NUM_NEG) + sos_sum / (2.0 * _N)
    out_ref[0, 0] += contrib


@jax.jit
def kernel(kp1, w_kp1, kp1_desc, desc2, homo12):
    b = kp1.shape[0]
    kxc = kp1[..., 0].reshape(b, _N, 1)
    kyc = kp1[..., 1].reshape(b, _N, 1)
    kxr = kp1[..., 0].reshape(b, 1, _N)
    kyr = kp1[..., 1].reshape(b, 1, _N)
    wxc = w_kp1[..., 0].reshape(b, _N, 1)
    wyc = w_kp1[..., 1].reshape(b, _N, 1)
    wxr = w_kp1[..., 0].reshape(b, 1, _N)
    wyr = w_kp1[..., 1].reshape(b, 1, _N)
    kxp = kp1[..., 0].reshape(b, 8, 128)
    kyp = kp1[..., 1].reshape(b, 8, 128)
    d2r = jnp.transpose(desc2, (0, 2, 3, 1)).reshape(b, _M, _C)
    homo = homo12.reshape(b, 1, 9)

    col3 = pl.BlockSpec((1, _N, 1), lambda i: (i, 0, 0))
    row3 = pl.BlockSpec((1, 1, _N), lambda i: (i, 0, 0))

    out = pl.pallas_call(
        _loss_kernel,
        grid=(b,),
        in_specs=[
            col3, col3, row3, row3,
            col3, col3, row3, row3,
            pl.BlockSpec((1, 8, 128), lambda i: (i, 0, 0)),
            pl.BlockSpec((1, 8, 128), lambda i: (i, 0, 0)),
            pl.BlockSpec((1, _N, _C), lambda i: (i, 0, 0)),
            pl.BlockSpec((1, _M, _C), lambda i: (i, 0, 0)),
            pl.BlockSpec((1, 1, 9), lambda i: (i, 0, 0),
                         memory_space=pltpu.SMEM),
        ],
        out_specs=pl.BlockSpec((1, 1), lambda i: (0, 0),
                               memory_space=pltpu.SMEM),
        out_shape=jax.ShapeDtypeStruct((1, 1), jnp.float32),
    )(kxc, kyc, kxr, kyr, wxc, wyc, wxr, wyr, kxp, kyp, kp1_desc, d2r, homo)
    return out[0, 0]


# speedup vs baseline: 1.8487x; 1.0461x over previous
import functools

import jax
import jax.numpy as jnp
from jax.experimental import pallas as pl
from jax.experimental.pallas import tpu as pltpu

_GRID_SIZE = 16.0
_MARGIN = 1.0
_NUM_NEG = 8
_SOS_NEG = 8
_N = 1024
_C = 256
_M = 1024
_BIG = 1.0e6
_RADIUS = _GRID_SIZE * (2.0 ** 0.5) + 0.1


def _row_min(x):
    return jnp.min(x, axis=1, keepdims=True)


def _treemin(xs):
    while len(xs) > 1:
        xs = [jnp.minimum(xs[i], xs[i + 1]) for i in range(0, len(xs) - 1, 2)
              ] + (xs[-1:] if len(xs) % 2 else [])
    return xs[0]


_NET8 = ((0, 1), (2, 3), (4, 5), (6, 7),
         (0, 2), (1, 3), (4, 6), (5, 7),
         (1, 2), (5, 6), (0, 4), (3, 7),
         (1, 5), (2, 6),
         (1, 4), (3, 6),
         (2, 4), (3, 5),
         (3, 4))


def _top8_stack(x):
    s = [x[:, 128 * k:128 * (k + 1)] for k in range(8)]
    for a, b in _NET8:
        lo = jnp.minimum(s[a], s[b])
        hi = jnp.maximum(s[a], s[b])
        s[a], s[b] = lo, hi
    out = []
    for _ in range(8):
        m = _row_min(s[0])
        out.append(m)
        cond = s[0] == m
        s = ([jnp.where(cond, s[j + 1], s[j]) for j in range(7)]
             + [jnp.where(cond, jnp.float32(jnp.inf), s[7])])
    return out


def _top4_axis(vals):
    out_v, out_i = [], []
    for _ in range(4):
        m = _treemin(vals)
        f = _treemin([jnp.where(v == m, jnp.float32(j), jnp.float32(1e9))
                      for j, v in enumerate(vals)])
        vals = [jnp.where((v == m) & (f == jnp.float32(j)),
                          jnp.float32(jnp.inf), v)
                for j, v in enumerate(vals)]
        out_v.append(m)
        out_i.append(f)
    return out_v, out_i


def _top4_cells_packed(px, py):
    dx2 = [(px - (j + 0.5) * _GRID_SIZE) ** 2 for j in range(32)]
    dy2 = [(py - (i + 0.5) * _GRID_SIZE) ** 2 for i in range(32)]
    vx, jx = _top4_axis(dx2)
    vy, iy = _top4_axis(dy2)
    ds = [vy[a] + vx[b] for a in range(4) for b in range(4)]
    fl = [iy[a] * 32.0 + jx[b] for a in range(4) for b in range(4)]
    out = []
    for _ in range(4):
        m = _treemin(ds)
        f = _treemin([jnp.where(d == m, g, jnp.float32(1e9))
                      for d, g in zip(ds, fl)])
        ds = [jnp.where((d == m) & (g == f), jnp.float32(jnp.inf), d)
              for d, g in zip(ds, fl)]
        out.append(f)
    return out


def _extract_min(x, cols_f):
    minv = _row_min(x)
    cand = jnp.where(x == minv, cols_f, jnp.float32(2.0 * _M))
    amin = _row_min(cand)
    x = jnp.where(cols_f == amin, jnp.float32(jnp.inf), x)
    return minv, amin, x


def _loss_kernel(kxc_ref, kyc_ref, kxr_ref, kyr_ref,
                 wxc_ref, wyc_ref, wxr_ref, wyr_ref,
                 kxp_ref, kyp_ref,
                 desc_ref, d2r_ref, homo_ref, out_ref):
    b = pl.program_id(0)

    @pl.when(b == 0)
    def _init():
        out_ref[0, 0] = jnp.float32(0.0)

    kxc = kxc_ref[0]
    kyc = kyc_ref[0]
    kxr = kxr_ref[0]
    kyr = kyr_ref[0]
    wxc = wxc_ref[0]
    wyc = wyc_ref[0]
    wxr = wxr_ref[0]
    wyr = wyr_ref[0]
    desc = desc_ref[0]
    d2r = d2r_ref[0]

    cols_f = jax.lax.broadcasted_iota(jnp.int32, (1, _M), 1).astype(
        jnp.float32)
    cols3 = jax.lax.broadcasted_iota(jnp.int32, (1, 1, _M), 2).astype(
        jnp.float32)
    cell_i = jnp.floor(cols_f * (1.0 / 32.0))
    cell_j = cols_f - 32.0 * cell_i
    cellx = (cell_j + 0.5) * _GRID_SIZE
    celly = (cell_i + 0.5) * _GRID_SIZE

    x = jnp.clip(wxc * (1.0 / _GRID_SIZE) - 0.5, 0.0, 31.0)
    y = jnp.clip(wyc * (1.0 / _GRID_SIZE) - 0.5, 0.0, 31.0)
    x0 = jnp.floor(x)
    y0 = jnp.floor(y)
    x1 = jnp.minimum(x0 + 1.0, 31.0)
    y1 = jnp.minimum(y0 + 1.0, 31.0)
    wx = x - x0
    wy = y - y0
    w00 = (1.0 - wy) * (1.0 - wx)
    w01 = (1.0 - wy) * wx
    w10 = wy * (1.0 - wx)
    w11 = wy * wx
    onehot = (w00 * (cols_f == y0 * 32.0 + x0) +
              w01 * (cols_f == y0 * 32.0 + x1) +
              w10 * (cols_f == y1 * 32.0 + x0) +
              w11 * (cols_f == y1 * 32.0 + x1))
    wdesc = jax.lax.dot_general(onehot, d2r, (((1,), (0,)), ((), ())),
                                preferred_element_type=jnp.float32)
    nrm = jnp.sqrt(jnp.sum(wdesc * wdesc, axis=1, keepdims=True))
    wdesc = wdesc / (nrm + 1e-8)

    pos = jnp.sqrt(jnp.clip(2.0 - 2.0 * jnp.sum(desc * wdesc, axis=1,
                                                keepdims=True), 1e-8))

    desc_sim2 = 2.0 - 2.0 * jax.lax.dot_general(
        desc, d2r, (((1,), (1,)), ((), ())),
        preferred_element_type=jnp.float32)

    h00 = homo_ref[0, 0, 0]
    h01 = homo_ref[0, 0, 1]
    h02 = homo_ref[0, 0, 2]
    h10 = homo_ref[0, 0, 3]
    h11 = homo_ref[0, 0, 4]
    h12 = homo_ref[0, 0, 5]
    h20 = homo_ref[0, 0, 6]
    h21 = homo_ref[0, 0, 7]
    h22 = homo_ref[0, 0, 8]

    kxp = kxp_ref[0]
    kyp = kyp_ref[0]
    mask_ids = []
    for f in _top4_cells_packed(kxp, kyp):
        ci = jnp.floor(f * (1.0 / 32.0))
        cj = f - 32.0 * ci
        cx = (cj + 0.5) * _GRID_SIZE
        cy = (ci + 0.5) * _GRID_SIZE
        den = h20 * cx + h21 * cy + h22 + 1e-8
        wcx = (h00 * cx + h01 * cy + h02) / den
        wcy = (h10 * cx + h11 * cy + h12) / den
        mask_ids.extend(_top4_cells_packed(wcx, wcy))
    macc = [(cols3 == f2[:, :, None]).astype(jnp.float32) for f2 in mask_ids]
    while len(macc) > 1:
        macc = [macc[i] + macc[i + 1] for i in range(0, len(macc), 2)]
    desc_sim2 = (desc_sim2.reshape(8, 128, _M)
                 + _BIG * macc[0]).reshape(_N, _M)

    fos_vec = jnp.zeros((_N, 1), jnp.float32)
    for minv in _top8_stack(desc_sim2):
        neg = jnp.sqrt(jnp.clip(minv, 1e-8))
        fos_vec = fos_vec + jnp.clip(pos - neg + _MARGIN, 0.0) ** 2
    fos_sum = jnp.sum(fos_vec)

    kp1_sim2 = 2.0 - 2.0 * jax.lax.dot_general(
        desc, desc, (((1,), (1,)), ((), ())),
        preferred_element_type=jnp.float32)
    kdist = jnp.sqrt((kxc - kxr) ** 2 + (kyc - kyr) ** 2 + 1e-8)
    kp1_sim2 = kp1_sim2 + _BIG * (kdist <= _RADIUS)

    w_sim2 = 2.0 - 2.0 * jax.lax.dot_general(
        wdesc, wdesc, (((1,), (1,)), ((), ())),
        preferred_element_type=jnp.float32)
    wdist = jnp.sqrt((wxc - wxr) ** 2 + (wyc - wyr) ** 2 + 1e-8)
    w_sim2 = w_sim2 + _BIG * (wdist <= _RADIUS)

    colsn_f = jax.lax.broadcasted_iota(jnp.int32, (1, _N), 1).astype(
        jnp.float32)
    sos_vec = jnp.zeros((_N, 1), jnp.float32)
    for mva, mvb in zip(_top8_stack(kp1_sim2), _top8_stack(w_sim2)):
        a = jnp.sqrt(jnp.clip(mva, 1e-8))
        bb = jnp.sqrt(jnp.clip(mvb, 1e-8))
        sos_vec = sos_vec + (a - bb) ** 2
    sos_sum = jnp.sum(jnp.sqrt(sos_vec + 1e-8))

    contrib = fos_sum / (2.0 * _N * _NUM_NEG) + sos_sum / (2.0 * _N)
    out_ref[0, 0] += contrib


@jax.jit
def kernel(kp1, w_kp1, kp1_desc, desc2, homo12):
    b = kp1.shape[0]
    kxc = kp1[..., 0].reshape(b, _N, 1)
    kyc = kp1[..., 1].reshape(b, _N, 1)
    kxr = kp1[..., 0].reshape(b, 1, _N)
    kyr = kp1[..., 1].reshape(b, 1, _N)
    wxc = w_kp1[..., 0].reshape(b, _N, 1)
    wyc = w_kp1[..., 1].reshape(b, _N, 1)
    wxr = w_kp1[..., 0].reshape(b, 1, _N)
    wyr = w_kp1[..., 1].reshape(b, 1, _N)
    kxp = kp1[..., 0].reshape(b, 8, 128)
    kyp = kp1[..., 1].reshape(b, 8, 128)
    d2r = jnp.transpose(desc2, (0, 2, 3, 1)).reshape(b, _M, _C)
    homo = homo12.reshape(b, 1, 9)

    col3 = pl.BlockSpec((1, _N, 1), lambda i: (i, 0, 0))
    row3 = pl.BlockSpec((1, 1, _N), lambda i: (i, 0, 0))

    out = pl.pallas_call(
        _loss_kernel,
        grid=(b,),
        in_specs=[
            col3, col3, row3, row3,
            col3, col3, row3, row3,
            pl.BlockSpec((1, 8, 128), lambda i: (i, 0, 0)),
            pl.BlockSpec((1, 8, 128), lambda i: (i, 0, 0)),
            pl.BlockSpec((1, _N, _C), lambda i: (i, 0, 0)),
            pl.BlockSpec((1, _M, _C), lambda i: (i, 0, 0)),
            pl.BlockSpec((1, 1, 9), lambda i: (i, 0, 0),
                         memory_space=pltpu.SMEM),
        ],
        out_specs=pl.BlockSpec((1, 1), lambda i: (0, 0),
                               memory_space=pltpu.SMEM),
        out_shape=jax.ShapeDtypeStruct((1, 1), jnp.float32),
    )(kxc, kyc, kxr, kyr, wxc, wyc, wxr, wyr, kxp, kyp, kp1_desc, d2r, homo)
    return out[0, 0]


# speedup vs baseline: 2.0674x; 1.1183x over previous
import functools

import jax
import jax.numpy as jnp
from jax.experimental import pallas as pl
from jax.experimental.pallas import tpu as pltpu

_GRID_SIZE = 16.0
_MARGIN = 1.0
_NUM_NEG = 8
_SOS_NEG = 8
_N = 1024
_C = 256
_M = 1024
_BIG = 1.0e6
_RADIUS = _GRID_SIZE * (2.0 ** 0.5) + 0.1


def _row_min(x):
    return jnp.min(x, axis=1, keepdims=True)


def _treemin(xs):
    while len(xs) > 1:
        xs = [jnp.minimum(xs[i], xs[i + 1]) for i in range(0, len(xs) - 1, 2)
              ] + (xs[-1:] if len(xs) % 2 else [])
    return xs[0]


_NET8 = ((0, 1), (2, 3), (4, 5), (6, 7),
         (0, 2), (1, 3), (4, 6), (5, 7),
         (1, 2), (5, 6), (0, 4), (3, 7),
         (1, 5), (2, 6),
         (1, 4), (3, 6),
         (2, 4), (3, 5),
         (3, 4))


def _top8_stack(x):
    s = [x[:, 128 * k:128 * (k + 1)] for k in range(8)]
    for a, b in _NET8:
        lo = jnp.minimum(s[a], s[b])
        hi = jnp.maximum(s[a], s[b])
        s[a], s[b] = lo, hi
    out = []
    for _ in range(8):
        m = _row_min(s[0])
        out.append(m)
        cond = s[0] == m
        s = ([jnp.where(cond, s[j + 1], s[j]) for j in range(7)]
             + [jnp.where(cond, jnp.float32(jnp.inf), s[7])])
    return out


def _top4_axis(vals):
    out_v, out_i = [], []
    for _ in range(4):
        m = _treemin(vals)
        f = _treemin([jnp.where(v == m, jnp.float32(j), jnp.float32(1e9))
                      for j, v in enumerate(vals)])
        vals = [jnp.where((v == m) & (f == jnp.float32(j)),
                          jnp.float32(jnp.inf), v)
                for j, v in enumerate(vals)]
        out_v.append(m)
        out_i.append(f)
    return out_v, out_i


def _top4_cells_packed(px, py):
    dx2 = [(px - (j + 0.5) * _GRID_SIZE) ** 2 for j in range(32)]
    dy2 = [(py - (i + 0.5) * _GRID_SIZE) ** 2 for i in range(32)]
    vx, jx = _top4_axis(dx2)
    vy, iy = _top4_axis(dy2)
    ds = [vy[a] + vx[b] for a in range(4) for b in range(4)]
    fl = [iy[a] * 32.0 + jx[b] for a in range(4) for b in range(4)]
    out = []
    for _ in range(4):
        m = _treemin(ds)
        f = _treemin([jnp.where(d == m, g, jnp.float32(1e9))
                      for d, g in zip(ds, fl)])
        ds = [jnp.where((d == m) & (g == f), jnp.float32(jnp.inf), d)
              for d, g in zip(ds, fl)]
        out.append(f)
    return out


def _extract_min(x, cols_f):
    minv = _row_min(x)
    cand = jnp.where(x == minv, cols_f, jnp.float32(2.0 * _M))
    amin = _row_min(cand)
    x = jnp.where(cols_f == amin, jnp.float32(jnp.inf), x)
    return minv, amin, x


def _loss_kernel(kxr_ref, kyr_ref, wxr_ref, wyr_ref,
                 kxp_ref, kyp_ref, wxp_ref, wyp_ref,
                 desc_ref, d2r_ref, homo_ref, out_ref):
    b = pl.program_id(0)

    @pl.when(b == 0)
    def _init():
        out_ref[0, 0] = jnp.float32(0.0)

    kxr = kxr_ref[0]
    kyr = kyr_ref[0]
    wxr = wxr_ref[0]
    wyr = wyr_ref[0]
    desc = desc_ref[0]
    d2r = d2r_ref[0]

    cols_f = jax.lax.broadcasted_iota(jnp.int32, (1, _M), 1).astype(
        jnp.float32)
    cols3 = jax.lax.broadcasted_iota(jnp.int32, (1, 1, _M), 2).astype(
        jnp.float32)
    cell_i = jnp.floor(cols_f * (1.0 / 32.0))
    cell_j = cols_f - 32.0 * cell_i
    cellx = (cell_j + 0.5) * _GRID_SIZE
    celly = (cell_i + 0.5) * _GRID_SIZE

    wxp = wxp_ref[0]
    wyp = wyp_ref[0]
    x = jnp.clip(wxp * (1.0 / _GRID_SIZE) - 0.5, 0.0, 31.0)[:, :, None]
    y = jnp.clip(wyp * (1.0 / _GRID_SIZE) - 0.5, 0.0, 31.0)[:, :, None]
    x0 = jnp.floor(x)
    y0 = jnp.floor(y)
    x1 = jnp.minimum(x0 + 1.0, 31.0)
    y1 = jnp.minimum(y0 + 1.0, 31.0)
    wx = x - x0
    wy = y - y0
    cell_i3 = jnp.floor(cols3 * (1.0 / 32.0))
    cell_j3 = cols3 - 32.0 * cell_i3
    prow = (cell_i3 == y0) * (1.0 - wy) + (cell_i3 == y1) * wy
    pcol = (cell_j3 == x0) * (1.0 - wx) + (cell_j3 == x1) * wx
    onehot = (prow * pcol).reshape(_N, _M)
    wdesc = jax.lax.dot_general(onehot, d2r, (((1,), (0,)), ((), ())),
                                preferred_element_type=jnp.float32)
    nrm = jnp.sqrt(jnp.sum(wdesc * wdesc, axis=1, keepdims=True))
    wdesc = wdesc / (nrm + 1e-8)

    pos = jnp.sqrt(jnp.clip(2.0 - 2.0 * jnp.sum(desc * wdesc, axis=1,
                                                keepdims=True), 1e-8))

    desc_sim2 = 2.0 - 2.0 * jax.lax.dot_general(
        desc, d2r, (((1,), (1,)), ((), ())),
        preferred_element_type=jnp.float32)

    h00 = homo_ref[0, 0, 0]
    h01 = homo_ref[0, 0, 1]
    h02 = homo_ref[0, 0, 2]
    h10 = homo_ref[0, 0, 3]
    h11 = homo_ref[0, 0, 4]
    h12 = homo_ref[0, 0, 5]
    h20 = homo_ref[0, 0, 6]
    h21 = homo_ref[0, 0, 7]
    h22 = homo_ref[0, 0, 8]

    kxp = kxp_ref[0]
    kyp = kyp_ref[0]
    mask_ids = []
    for f in _top4_cells_packed(kxp, kyp):
        ci = jnp.floor(f * (1.0 / 32.0))
        cj = f - 32.0 * ci
        cx = (cj + 0.5) * _GRID_SIZE
        cy = (ci + 0.5) * _GRID_SIZE
        den = h20 * cx + h21 * cy + h22 + 1e-8
        wcx = (h00 * cx + h01 * cy + h02) / den
        wcy = (h10 * cx + h11 * cy + h12) / den
        mask_ids.extend(_top4_cells_packed(wcx, wcy))
    macc = [(cols3 == f2[:, :, None]).astype(jnp.float32) for f2 in mask_ids]
    while len(macc) > 1:
        macc = [macc[i] + macc[i + 1] for i in range(0, len(macc), 2)]
    desc_sim2 = (desc_sim2.reshape(8, 128, _M)
                 + _BIG * macc[0]).reshape(_N, _M)

    fos_vec = jnp.zeros((_N, 1), jnp.float32)
    for minv in _top8_stack(desc_sim2):
        neg = jnp.sqrt(jnp.clip(minv, 1e-8))
        fos_vec = fos_vec + jnp.clip(pos - neg + _MARGIN, 0.0) ** 2
    fos_sum = jnp.sum(fos_vec)

    kp1_sim2 = 2.0 - 2.0 * jax.lax.dot_general(
        desc, desc, (((1,), (1,)), ((), ())),
        preferred_element_type=jnp.float32)
    kxr3 = kxr[None]
    kyr3 = kyr[None]
    kdist = jnp.sqrt((kxp[:, :, None] - kxr3) ** 2
                     + (kyp[:, :, None] - kyr3) ** 2 + 1e-8)
    kp1_sim2 = (kp1_sim2.reshape(8, 128, _N)
                + _BIG * (kdist <= _RADIUS)).reshape(_N, _N)

    w_sim2 = 2.0 - 2.0 * jax.lax.dot_general(
        wdesc, wdesc, (((1,), (1,)), ((), ())),
        preferred_element_type=jnp.float32)
    wdist = jnp.sqrt((wxp[:, :, None] - wxr[None]) ** 2
                     + (wyp[:, :, None] - wyr[None]) ** 2 + 1e-8)
    w_sim2 = (w_sim2.reshape(8, 128, _N)
              + _BIG * (wdist <= _RADIUS)).reshape(_N, _N)

    colsn_f = jax.lax.broadcasted_iota(jnp.int32, (1, _N), 1).astype(
        jnp.float32)
    sos_vec = jnp.zeros((_N, 1), jnp.float32)
    for mva, mvb in zip(_top8_stack(kp1_sim2), _top8_stack(w_sim2)):
        a = jnp.sqrt(jnp.clip(mva, 1e-8))
        bb = jnp.sqrt(jnp.clip(mvb, 1e-8))
        sos_vec = sos_vec + (a - bb) ** 2
    sos_sum = jnp.sum(jnp.sqrt(sos_vec + 1e-8))

    contrib = fos_sum / (2.0 * _N * _NUM_NEG) + sos_sum / (2.0 * _N)
    out_ref[0, 0] += contrib


@jax.jit
def kernel(kp1, w_kp1, kp1_desc, desc2, homo12):
    b = kp1.shape[0]
    kxr = kp1[..., 0].reshape(b, 1, _N)
    kyr = kp1[..., 1].reshape(b, 1, _N)
    wxr = w_kp1[..., 0].reshape(b, 1, _N)
    wyr = w_kp1[..., 1].reshape(b, 1, _N)
    kxp = kp1[..., 0].reshape(b, 8, 128)
    kyp = kp1[..., 1].reshape(b, 8, 128)
    wxp = w_kp1[..., 0].reshape(b, 8, 128)
    wyp = w_kp1[..., 1].reshape(b, 8, 128)
    d2r = jnp.transpose(desc2, (0, 2, 3, 1)).reshape(b, _M, _C)
    homo = homo12.reshape(b, 1, 9)

    row3 = pl.BlockSpec((1, 1, _N), lambda i: (i, 0, 0))
    pk3 = pl.BlockSpec((1, 8, 128), lambda i: (i, 0, 0))

    out = pl.pallas_call(
        _loss_kernel,
        grid=(b,),
        in_specs=[
            row3, row3, row3, row3,
            pk3, pk3, pk3, pk3,
            pl.BlockSpec((1, _N, _C), lambda i: (i, 0, 0)),
            pl.BlockSpec((1, _M, _C), lambda i: (i, 0, 0)),
            pl.BlockSpec((1, 1, 9), lambda i: (i, 0, 0),
                         memory_space=pltpu.SMEM),
        ],
        out_specs=pl.BlockSpec((1, 1), lambda i: (0, 0),
                               memory_space=pltpu.SMEM),
        out_shape=jax.ShapeDtypeStruct((1, 1), jnp.float32),
    )(kxr, kyr, wxr, wyr, kxp, kyp, wxp, wyp, kp1_desc, d2r, homo)
    return out[0, 0]


# speedup vs baseline: 2.1230x; 1.0269x over previous
import functools

import jax
import jax.numpy as jnp
from jax.experimental import pallas as pl
from jax.experimental.pallas import tpu as pltpu

_GRID_SIZE = 16.0
_MARGIN = 1.0
_NUM_NEG = 8
_SOS_NEG = 8
_N = 1024
_C = 256
_M = 1024
_BIG = 1.0e6
_RADIUS = _GRID_SIZE * (2.0 ** 0.5) + 0.1
_RADIUS2 = _RADIUS * _RADIUS


def _row_min(x):
    return jnp.min(x, axis=1, keepdims=True)


def _treemin(xs):
    while len(xs) > 1:
        xs = [jnp.minimum(xs[i], xs[i + 1]) for i in range(0, len(xs) - 1, 2)
              ] + (xs[-1:] if len(xs) % 2 else [])
    return xs[0]


_NET8 = ((0, 1), (2, 3), (4, 5), (6, 7),
         (0, 2), (1, 3), (4, 6), (5, 7),
         (1, 2), (5, 6), (0, 4), (3, 7),
         (1, 5), (2, 6),
         (1, 4), (3, 6),
         (2, 4), (3, 5),
         (3, 4))


def _top8_stack(x):
    s = [x[:, 128 * k:128 * (k + 1)] for k in range(8)]
    for a, b in _NET8:
        lo = jnp.minimum(s[a], s[b])
        hi = jnp.maximum(s[a], s[b])
        s[a], s[b] = lo, hi
    out = []
    for _ in range(8):
        m = _row_min(s[0])
        out.append(m)
        cond = s[0] == m
        s = ([jnp.where(cond, s[j + 1], s[j]) for j in range(7)]
             + [jnp.where(cond, jnp.float32(jnp.inf), s[7])])
    return out


def _top4_axis(vals):
    out_v, out_i = [], []
    for _ in range(4):
        m = _treemin(vals)
        f = _treemin([jnp.where(v == m, jnp.float32(j), jnp.float32(1e9))
                      for j, v in enumerate(vals)])
        vals = [jnp.where((v == m) & (f == jnp.float32(j)),
                          jnp.float32(jnp.inf), v)
                for j, v in enumerate(vals)]
        out_v.append(m)
        out_i.append(f)
    return out_v, out_i


def _top4_cells_packed(px, py):
    dx2 = [(px - (j + 0.5) * _GRID_SIZE) ** 2 for j in range(32)]
    dy2 = [(py - (i + 0.5) * _GRID_SIZE) ** 2 for i in range(32)]
    vx, jx = _top4_axis(dx2)
    vy, iy = _top4_axis(dy2)
    ds = [vy[a] + vx[b] for a in range(4) for b in range(4)]
    fl = [iy[a] * 32.0 + jx[b] for a in range(4) for b in range(4)]
    out = []
    for _ in range(4):
        m = _treemin(ds)
        f = _treemin([jnp.where(d == m, g, jnp.float32(1e9))
                      for d, g in zip(ds, fl)])
        ds = [jnp.where((d == m) & (g == f), jnp.float32(jnp.inf), d)
              for d, g in zip(ds, fl)]
        out.append(f)
    return out


def _extract_min(x, cols_f):
    minv = _row_min(x)
    cand = jnp.where(x == minv, cols_f, jnp.float32(2.0 * _M))
    amin = _row_min(cand)
    x = jnp.where(cols_f == amin, jnp.float32(jnp.inf), x)
    return minv, amin, x


def _loss_kernel(kxr_ref, kyr_ref, wxr_ref, wyr_ref,
                 kxp_ref, kyp_ref, wxp_ref, wyp_ref,
                 desc_ref, d2r_ref, homo_ref, out_ref):
    b = pl.program_id(0)

    @pl.when(b == 0)
    def _init():
        out_ref[0, 0] = jnp.float32(0.0)

    kxr = kxr_ref[0]
    kyr = kyr_ref[0]
    wxr = wxr_ref[0]
    wyr = wyr_ref[0]
    desc = desc_ref[0]
    d2r = d2r_ref[0]

    cols_f = jax.lax.broadcasted_iota(jnp.int32, (1, _M), 1).astype(
        jnp.float32)
    cols3 = jax.lax.broadcasted_iota(jnp.int32, (1, 1, _M), 2).astype(
        jnp.float32)
    cell_i = jnp.floor(cols_f * (1.0 / 32.0))
    cell_j = cols_f - 32.0 * cell_i
    cellx = (cell_j + 0.5) * _GRID_SIZE
    celly = (cell_i + 0.5) * _GRID_SIZE

    wxp = wxp_ref[0]
    wyp = wyp_ref[0]
    x = jnp.clip(wxp * (1.0 / _GRID_SIZE) - 0.5, 0.0, 31.0)[:, :, None]
    y = jnp.clip(wyp * (1.0 / _GRID_SIZE) - 0.5, 0.0, 31.0)[:, :, None]
    x0 = jnp.floor(x)
    y0 = jnp.floor(y)
    x1 = jnp.minimum(x0 + 1.0, 31.0)
    y1 = jnp.minimum(y0 + 1.0, 31.0)
    wx = x - x0
    wy = y - y0
    cell_i3 = jnp.floor(cols3 * (1.0 / 32.0))
    cell_j3 = cols3 - 32.0 * cell_i3
    prow = (cell_i3 == y0) * (1.0 - wy) + (cell_i3 == y1) * wy
    pcol = (cell_j3 == x0) * (1.0 - wx) + (cell_j3 == x1) * wx
    onehot = (prow * pcol).reshape(_N, _M)
    wdesc = jax.lax.dot_general(onehot, d2r, (((1,), (0,)), ((), ())),
                                preferred_element_type=jnp.float32)
    nrm = jnp.sqrt(jnp.sum(wdesc * wdesc, axis=1, keepdims=True))
    wdesc = wdesc / (nrm + 1e-8)

    pos = jnp.sqrt(jnp.clip(2.0 - 2.0 * jnp.sum(desc * wdesc, axis=1,
                                                keepdims=True), 1e-8))

    descm = -2.0 * desc
    desc_sim2 = jax.lax.dot_general(
        descm, d2r, (((1,), (1,)), ((), ())),
        preferred_element_type=jnp.float32)

    h00 = homo_ref[0, 0, 0]
    h01 = homo_ref[0, 0, 1]
    h02 = homo_ref[0, 0, 2]
    h10 = homo_ref[0, 0, 3]
    h11 = homo_ref[0, 0, 4]
    h12 = homo_ref[0, 0, 5]
    h20 = homo_ref[0, 0, 6]
    h21 = homo_ref[0, 0, 7]
    h22 = homo_ref[0, 0, 8]

    kxp = kxp_ref[0]
    kyp = kyp_ref[0]
    mask_ids = []
    for f in _top4_cells_packed(kxp, kyp):
        ci = jnp.floor(f * (1.0 / 32.0))
        cj = f - 32.0 * ci
        cx = (cj + 0.5) * _GRID_SIZE
        cy = (ci + 0.5) * _GRID_SIZE
        den = h20 * cx + h21 * cy + h22 + 1e-8
        wcx = (h00 * cx + h01 * cy + h02) / den
        wcy = (h10 * cx + h11 * cy + h12) / den
        mask_ids.extend(_top4_cells_packed(wcx, wcy))
    macc = [cols3 == f2[:, :, None] for f2 in mask_ids]
    while len(macc) > 1:
        macc = [macc[i] | macc[i + 1] for i in range(0, len(macc), 2)]
    sim3 = desc_sim2.reshape(8, 128, _M)
    desc_sim2 = jnp.where(macc[0], sim3 + _BIG, sim3).reshape(_N, _M)

    fos_vec = jnp.zeros((_N, 1), jnp.float32)
    for minv in _top8_stack(desc_sim2):
        neg = jnp.sqrt(jnp.clip(minv + 2.0, 1e-8))
        fos_vec = fos_vec + jnp.clip(pos - neg + _MARGIN, 0.0) ** 2
    fos_sum = jnp.sum(fos_vec)

    kp1_sim2 = jax.lax.dot_general(
        descm, desc, (((1,), (1,)), ((), ())),
        preferred_element_type=jnp.float32)
    kxr3 = kxr[None]
    kyr3 = kyr[None]
    kd2 = ((kxp[:, :, None] - kxr3) ** 2
           + (kyp[:, :, None] - kyr3) ** 2 + 1e-8)
    ks3 = kp1_sim2.reshape(8, 128, _N)
    kp1_sim2 = jnp.where(kd2 <= _RADIUS2, ks3 + _BIG, ks3).reshape(_N, _N)

    w_sim2 = jax.lax.dot_general(
        -2.0 * wdesc, wdesc, (((1,), (1,)), ((), ())),
        preferred_element_type=jnp.float32)
    wd2 = ((wxp[:, :, None] - wxr[None]) ** 2
           + (wyp[:, :, None] - wyr[None]) ** 2 + 1e-8)
    ws3 = w_sim2.reshape(8, 128, _N)
    w_sim2 = jnp.where(wd2 <= _RADIUS2, ws3 + _BIG, ws3).reshape(_N, _N)

    colsn_f = jax.lax.broadcasted_iota(jnp.int32, (1, _N), 1).astype(
        jnp.float32)
    sos_vec = jnp.zeros((_N, 1), jnp.float32)
    for mva, mvb in zip(_top8_stack(kp1_sim2), _top8_stack(w_sim2)):
        a = jnp.sqrt(jnp.clip(mva + 2.0, 1e-8))
        bb = jnp.sqrt(jnp.clip(mvb + 2.0, 1e-8))
        sos_vec = sos_vec + (a - bb) ** 2
    sos_sum = jnp.sum(jnp.sqrt(sos_vec + 1e-8))

    contrib = fos_sum / (2.0 * _N * _NUM_NEG) + sos_sum / (2.0 * _N)
    out_ref[0, 0] += contrib


@jax.jit
def kernel(kp1, w_kp1, kp1_desc, desc2, homo12):
    b = kp1.shape[0]
    kxr = kp1[..., 0].reshape(b, 1, _N)
    kyr = kp1[..., 1].reshape(b, 1, _N)
    wxr = w_kp1[..., 0].reshape(b, 1, _N)
    wyr = w_kp1[..., 1].reshape(b, 1, _N)
    kxp = kp1[..., 0].reshape(b, 8, 128)
    kyp = kp1[..., 1].reshape(b, 8, 128)
    wxp = w_kp1[..., 0].reshape(b, 8, 128)
    wyp = w_kp1[..., 1].reshape(b, 8, 128)
    d2r = jnp.transpose(desc2, (0, 2, 3, 1)).reshape(b, _M, _C)
    homo = homo12.reshape(b, 1, 9)

    row3 = pl.BlockSpec((1, 1, _N), lambda i: (i, 0, 0))
    pk3 = pl.BlockSpec((1, 8, 128), lambda i: (i, 0, 0))

    out = pl.pallas_call(
        _loss_kernel,
        grid=(b,),
        in_specs=[
            row3, row3, row3, row3,
            pk3, pk3, pk3, pk3,
            pl.BlockSpec((1, _N, _C), lambda i: (i, 0, 0)),
            pl.BlockSpec((1, _M, _C), lambda i: (i, 0, 0)),
            pl.BlockSpec((1, 1, 9), lambda i: (i, 0, 0),
                         memory_space=pltpu.SMEM),
        ],
        out_specs=pl.BlockSpec((1, 1), lambda i: (0, 0),
                               memory_space=pltpu.SMEM),
        out_shape=jax.ShapeDtypeStruct((1, 1), jnp.float32),
    )(kxr, kyr, wxr, wyr, kxp, kyp, wxp, wyp, kp1_desc, d2r, homo)
    return out[0, 0]
